# baseline (speedup 1.0000x reference)
import functools

import jax
import jax.numpy as jnp
from jax import lax
from jax.experimental import pallas as pl
from jax.experimental.pallas import tpu as pltpu
from jax.experimental.pallas import tpu_sc as plsc

NWORDS = 1000000
NTAGS = 16
SEQ = 16384

NC = 2
NS = 16
NW = NC * NS
L = 16

FULLBLKS = NWORDS // 128
PADCOLS = FULLBLKS * 128 + 128
CHUNK = 2048
NCH = 16
CLOC = 31488
NIDXV = SEQ // L


def _bow_body(tablet_hbm, xflat_hbm, out_hbm, idx_v, c_loc, buf0, buf1,
              tmat, stage_v, sem):
    c = lax.axis_index("c")
    s = lax.axis_index("s")
    wid = s * NC + c

    f_blk = (FULLBLKS * wid) // 32
    e_blk = (FULLBLKS * (wid + 1)) // 32
    lo = f_blk * 128
    is_last = wid == NW - 1
    hi = jnp.where(is_last, NWORDS, e_blk * 128)
    hi_pad = jnp.where(is_last, PADCOLS, e_blk * 128)
    span = hi - lo
    pad_span = hi_pad - lo

    def chunk_start(k):
        return lo + jnp.minimum(k * CHUNK, pad_span - CHUNK)

    bufs = (buf0, buf1)

    def fire(k):
        sk = pl.multiple_of(chunk_start(k), 128)
        pltpu.async_copy(tablet_hbm.at[:, pl.ds(sk, CHUNK)], bufs[k % 2], sem)

    fire(0)
    fire(1)

    pltpu.sync_copy(xflat_hbm, idx_v)

    def zero_body(z, carry):
        c_loc[pl.ds(pl.multiple_of(z * L, L), L)] = jnp.zeros((L,), jnp.float32)
        return carry

    lax.fori_loop(0, CLOC // L, zero_body, 0, unroll=8)

    ones = jnp.ones((L,), jnp.float32)

    def count_body(v, carry):
        xv = idx_v[pl.ds(pl.multiple_of(v * L, L), L)]
        rel = xv - lo
        mask = (rel >= 0) & (rel < span)
        plsc.addupdate_scatter(c_loc, [rel], ones, mask=mask)
        return carry

    lax.fori_loop(0, NIDXV, count_body, 0, unroll=8)

    zacc = tuple(jnp.zeros((L,), jnp.float32) for _ in range(NTAGS))
    acc = zacc

    for k in range(NCH):
        buf = bufs[k % 2]
        sk = pl.multiple_of(chunk_start(k), 128)
        pltpu.make_async_copy(
            tablet_hbm.at[:, pl.ds(sk, CHUNK)], buf, sem
        ).wait()
        base_rel = pl.multiple_of(sk - lo, L)

        def group_body(g, a, buf=buf, base_rel=base_rel):
            off = pl.multiple_of(base_rel + g * L, L)
            cv = c_loc[pl.ds(off, L)]
            nz = jnp.max(cv)

            def hit():
                c_loc[pl.ds(off, L)] = jnp.zeros((L,), jnp.float32)
                go = pl.multiple_of(g * L, L)
                return tuple(
                    a[t] + buf[t, pl.ds(go, L)] * cv for t in range(NTAGS)
                )

            def miss():
                return a

            return lax.cond(nz > 0.0, hit, miss)

        acc = lax.fori_loop(0, CHUNK // L, group_body, acc, unroll=4)
        if k + 2 < NCH:
            fire(k + 2)

    for t in range(NTAGS):
        tmat[t, :] = acc[t]

    rowi = lax.iota(jnp.int32, L)
    red = jnp.zeros((L,), jnp.float32)
    for j in range(L):
        red = red + plsc.load_gather(tmat, [rowi, jnp.full((L,), j, jnp.int32)])
    stage_v[...] = red
    pltpu.sync_copy(stage_v, out_hbm.at[wid])


_bow_sc = functools.partial(
    pl.kernel,
    out_type=jax.ShapeDtypeStruct((NW, L), jnp.float32),
    mesh=plsc.VectorSubcoreMesh(core_axis_name="c", subcore_axis_name="s"),
    scratch_types=[
        pltpu.VMEM((SEQ,), jnp.int32),
        pltpu.VMEM((CLOC,), jnp.float32),
        pltpu.VMEM((NTAGS, CHUNK), jnp.float32),
        pltpu.VMEM((NTAGS, CHUNK), jnp.float32),
        pltpu.VMEM((L, L), jnp.float32),
        pltpu.VMEM((L,), jnp.float32),
        pltpu.SemaphoreType.DMA,
    ],
    compiler_params=pltpu.CompilerParams(needs_layout_passes=False),
)(_bow_body)


def kernel(x, table, bias):
    xi = x.astype(jnp.int32)
    tablet = table.T
    partials = _bow_sc(tablet, xi)
    return (jnp.sum(partials, axis=0) + bias).reshape(1, -1)


# speedup vs baseline: 1.1815x; 1.1815x over previous
import functools

import jax
import jax.numpy as jnp
from jax import lax
from jax.experimental import pallas as pl
from jax.experimental.pallas import tpu as pltpu
from jax.experimental.pallas import tpu_sc as plsc

NWORDS = 1000000
NTAGS = 16
SEQ = 16384

NC = 2
NS = 16
NW = NC * NS
L = 16

FULLBLKS = NWORDS // 128
PADCOLS = FULLBLKS * 128 + 128
CHUNK = 2048
NCH = 16
CLOC = 31488
NIDXV = SEQ // L


def _bow_body(tablet_hbm, xflat_hbm, out_hbm, idx_v, c_loc, buf0, buf1,
              tmat, stage_v, sem):
    c = lax.axis_index("c")
    s = lax.axis_index("s")
    wid = s * NC + c

    f_blk = (FULLBLKS * wid) // 32
    e_blk = (FULLBLKS * (wid + 1)) // 32
    lo = f_blk * 128
    is_last = wid == NW - 1
    hi = jnp.where(is_last, NWORDS, e_blk * 128)
    hi_pad = jnp.where(is_last, PADCOLS, e_blk * 128)
    span = hi - lo
    pad_span = hi_pad - lo

    def chunk_start(k):
        if k < NCH - 1:
            return lo + k * CHUNK
        return lo + pad_span - CHUNK

    bufs = (buf0, buf1)

    def fire(k):
        sk = pl.multiple_of(chunk_start(k), 128)
        pltpu.async_copy(tablet_hbm.at[:, pl.ds(sk, CHUNK)], bufs[k % 2], sem)

    fire(0)
    fire(1)

    pltpu.sync_copy(xflat_hbm, idx_v)

    def zero_body(z, carry):
        c_loc[pl.ds(pl.multiple_of(z * L, L), L)] = jnp.zeros((L,), jnp.float32)
        return carry

    lax.fori_loop(0, CLOC // L, zero_body, 0, unroll=8)

    ones = jnp.ones((L,), jnp.float32)

    def count_body(v, carry):
        xv = idx_v[pl.ds(pl.multiple_of(v * L, L), L)]
        rel = xv - lo
        mask = (rel >= 0) & (rel < span)
        plsc.addupdate_scatter(c_loc, [rel], ones, mask=mask)
        return carry

    lax.fori_loop(0, NIDXV, count_body, 0, unroll=8)

    zacc = tuple(jnp.zeros((L,), jnp.float32) for _ in range(NTAGS))
    acc = zacc

    for k in range(NCH):
        buf = bufs[k % 2]
        sk = pl.multiple_of(chunk_start(k), 128)
        pltpu.make_async_copy(
            tablet_hbm.at[:, pl.ds(sk, CHUNK)], buf, sem
        ).wait()
        base_rel = pl.multiple_of(sk - lo, L)
        g0 = (k * CHUNK - (sk - lo)) // L

        def group_body(g, a, buf=buf, base_rel=base_rel):
            off = pl.multiple_of(base_rel + g * L, L)
            cv = c_loc[pl.ds(off, L)]
            nz = jnp.max(cv)

            def hit():
                go = pl.multiple_of(g * L, L)
                return tuple(
                    a[t] + buf[t, pl.ds(go, L)] * cv for t in range(NTAGS)
                )

            def miss():
                return a

            return lax.cond(nz > 0.0, hit, miss)

        if k < NCH - 1:
            acc = lax.fori_loop(0, CHUNK // L, group_body, acc, unroll=2)
        else:
            acc = lax.fori_loop(g0, CHUNK // L, group_body, acc)
        if k + 2 < NCH:
            fire(k + 2)

    for t in range(NTAGS):
        tmat[t, :] = acc[t]

    rowi = lax.iota(jnp.int32, L)
    red = jnp.zeros((L,), jnp.float32)
    for j in range(L):
        red = red + plsc.load_gather(tmat, [rowi, jnp.full((L,), j, jnp.int32)])
    stage_v[...] = red
    pltpu.sync_copy(stage_v, out_hbm.at[wid])


_bow_sc = functools.partial(
    pl.kernel,
    out_type=jax.ShapeDtypeStruct((NW, L), jnp.float32),
    mesh=plsc.VectorSubcoreMesh(core_axis_name="c", subcore_axis_name="s"),
    scratch_types=[
        pltpu.VMEM((SEQ,), jnp.int32),
        pltpu.VMEM((CLOC,), jnp.float32),
        pltpu.VMEM((NTAGS, CHUNK), jnp.float32),
        pltpu.VMEM((NTAGS, CHUNK), jnp.float32),
        pltpu.VMEM((L, L), jnp.float32),
        pltpu.VMEM((L,), jnp.float32),
        pltpu.SemaphoreType.DMA,
    ],
    compiler_params=pltpu.CompilerParams(needs_layout_passes=False),
)(_bow_body)


def kernel(x, table, bias):
    xi = x.astype(jnp.int32)
    tablet = table.T
    partials = _bow_sc(tablet, xi)
    return (jnp.sum(partials, axis=0) + bias).reshape(1, -1)


# speedup vs baseline: 1.1984x; 1.0143x over previous
import functools

import jax
import jax.numpy as jnp
from jax import lax
from jax.experimental import pallas as pl
from jax.experimental.pallas import tpu as pltpu
from jax.experimental.pallas import tpu_sc as plsc

NWORDS = 1000000
NTAGS = 16
SEQ = 16384

NC = 2
NS = 16
NW = NC * NS
L = 16

FULLBLKS = NWORDS // 128
PADCOLS = FULLBLKS * 128 + 128
CHUNK = 1024
NCH = 31
NBUF = 4
CLOC = 31488
NIDXV = SEQ // L


def _bow_body(tablet_hbm, xflat_hbm, out_hbm, idx_v, c_loc, buf0, buf1,
              buf2, buf3, tmat, stage_v, sem):
    c = lax.axis_index("c")
    s = lax.axis_index("s")
    wid = s * NC + c

    f_blk = (FULLBLKS * wid) // 32
    e_blk = (FULLBLKS * (wid + 1)) // 32
    lo = f_blk * 128
    is_last = wid == NW - 1
    hi = jnp.where(is_last, NWORDS, e_blk * 128)
    hi_pad = jnp.where(is_last, PADCOLS, e_blk * 128)
    span = hi - lo
    pad_span = hi_pad - lo

    def chunk_start(k):
        if k < NCH - 1:
            return lo + k * CHUNK
        return lo + pad_span - CHUNK

    bufs = (buf0, buf1, buf2, buf3)

    def fire(k):
        sk = pl.multiple_of(chunk_start(k), 128)
        pltpu.async_copy(tablet_hbm.at[:, pl.ds(sk, CHUNK)], bufs[k % NBUF], sem)

    for _k in range(NBUF):
        fire(_k)

    pltpu.sync_copy(xflat_hbm, idx_v)

    def zero_body(z, carry):
        c_loc[pl.ds(pl.multiple_of(z * L, L), L)] = jnp.zeros((L,), jnp.float32)
        return carry

    lax.fori_loop(0, CLOC // L, zero_body, 0, unroll=8)

    ones = jnp.ones((L,), jnp.float32)

    def count_body(v, carry):
        xv = idx_v[pl.ds(pl.multiple_of(v * L, L), L)]
        rel = xv - lo
        mask = (rel >= 0) & (rel < span)
        plsc.addupdate_scatter(c_loc, [rel], ones, mask=mask)
        return carry

    lax.fori_loop(0, NIDXV, count_body, 0, unroll=8)

    zacc = tuple(jnp.zeros((L,), jnp.float32) for _ in range(NTAGS))
    acc = zacc

    for k in range(NCH):
        buf = bufs[k % NBUF]
        sk = pl.multiple_of(chunk_start(k), 128)
        pltpu.make_async_copy(
            tablet_hbm.at[:, pl.ds(sk, CHUNK)], buf, sem
        ).wait()
        base_rel = pl.multiple_of(sk - lo, L)
        g0 = (k * CHUNK - (sk - lo)) // L

        def group_body(g, a, buf=buf, base_rel=base_rel):
            off = pl.multiple_of(base_rel + g * L, L)
            cv = c_loc[pl.ds(off, L)]
            nz = jnp.max(cv)

            def hit():
                go = pl.multiple_of(g * L, L)
                return tuple(
                    a[t] + buf[t, pl.ds(go, L)] * cv for t in range(NTAGS)
                )

            def miss():
                return a

            return lax.cond(nz > 0.0, hit, miss)

        if k < NCH - 1:
            acc = lax.fori_loop(0, CHUNK // L, group_body, acc, unroll=2)
        else:
            acc = lax.fori_loop(g0, CHUNK // L, group_body, acc)
        if k + NBUF < NCH:
            fire(k + NBUF)

    for t in range(NTAGS):
        tmat[t, :] = acc[t]

    rowi = lax.iota(jnp.int32, L)
    red = jnp.zeros((L,), jnp.float32)
    for j in range(L):
        red = red + plsc.load_gather(tmat, [rowi, jnp.full((L,), j, jnp.int32)])
    stage_v[...] = red
    pltpu.sync_copy(stage_v, out_hbm.at[wid])


_bow_sc = functools.partial(
    pl.kernel,
    out_type=jax.ShapeDtypeStruct((NW, L), jnp.float32),
    mesh=plsc.VectorSubcoreMesh(core_axis_name="c", subcore_axis_name="s"),
    scratch_types=[
        pltpu.VMEM((SEQ,), jnp.int32),
        pltpu.VMEM((CLOC,), jnp.float32),
        pltpu.VMEM((NTAGS, CHUNK), jnp.float32),
        pltpu.VMEM((NTAGS, CHUNK), jnp.float32),
        pltpu.VMEM((NTAGS, CHUNK), jnp.float32),
        pltpu.VMEM((NTAGS, CHUNK), jnp.float32),
        pltpu.VMEM((L, L), jnp.float32),
        pltpu.VMEM((L,), jnp.float32),
        pltpu.SemaphoreType.DMA,
    ],
    compiler_params=pltpu.CompilerParams(needs_layout_passes=False),
)(_bow_body)


def kernel(x, table, bias):
    xi = x.astype(jnp.int32)
    tablet = table.T
    partials = _bow_sc(tablet, xi)
    return (jnp.sum(partials, axis=0) + bias).reshape(1, -1)
